# Initial kernel scaffold; baseline (speedup 1.0000x reference)
import jax
import jax.numpy as jnp
from jax.experimental import pallas as pl


def kernel(h, e, edge_index, params):
    raise NotImplementedError("write your pallas kernel here")


# speedup vs baseline: 2.5084x; 2.5084x over previous
import functools

import jax
import jax.numpy as jnp
from jax import lax
from jax.experimental import pallas as pl
from jax.experimental.pallas import tpu as pltpu
from jax.experimental.pallas import tpu_sc as plsc

_N = 10000
_E = 320000
_H = 128
_NC = 2
_NS = 16
_NW = _NC * _NS
_CH = 80

_f32 = jnp.float32


def _dot(a, b):
    return jnp.dot(a, b, preferred_element_type=_f32)


def _mm_bias_kernel(x_ref, w_ref, b_ref, o_ref):
    o_ref[...] = _dot(x_ref[...], w_ref[...]) + b_ref[...]


def _mm(x, w, b, blk):
    m, k = x.shape
    n = w.shape[1]
    return pl.pallas_call(
        _mm_bias_kernel,
        grid=(m // blk,),
        in_specs=[
            pl.BlockSpec((blk, k), lambda i: (i, 0)),
            pl.BlockSpec((k, n), lambda i: (0, 0)),
            pl.BlockSpec((1, n), lambda i: (0, 0)),
        ],
        out_specs=pl.BlockSpec((blk, n), lambda i: (i, 0)),
        out_shape=jax.ShapeDtypeStruct((m, n), _f32),
    )(x, w, b)


def _node_tf_kernel(h_ref, wa, ba, wb, bb, wd, bd, we, be,
                    ah_ref, db_ref, eh_ref):
    h = h_ref[...]
    ah_ref[...] = _dot(h, wa[...]) + ba[...]
    db_ref[:, :_H] = _dot(h, wd[...]) + bd[...]
    db_ref[:, _H:] = _dot(h, wb[...]) + bb[...]
    eh_ref[...] = _dot(h, we[...]) + be[...]


def _node_tf(h, lp, blk=2000):
    wspec = pl.BlockSpec((_H, _H), lambda i: (0, 0))
    bspec = pl.BlockSpec((1, _H), lambda i: (0, 0))
    r2 = lambda b: b.reshape(1, _H)
    return pl.pallas_call(
        _node_tf_kernel,
        grid=(_N // blk,),
        in_specs=[pl.BlockSpec((blk, _H), lambda i: (i, 0))]
        + [wspec, bspec] * 4,
        out_specs=[
            pl.BlockSpec((blk, _H), lambda i: (i, 0)),
            pl.BlockSpec((blk, 2 * _H), lambda i: (i, 0)),
            pl.BlockSpec((blk, _H), lambda i: (i, 0)),
        ],
        out_shape=[
            jax.ShapeDtypeStruct((_N, _H), _f32),
            jax.ShapeDtypeStruct((_N, 2 * _H), _f32),
            jax.ShapeDtypeStruct((_N, _H), _f32),
        ],
    )(h, lp['A'][0], r2(lp['A'][1]), lp['B'][0], r2(lp['B'][1]),
      lp['D'][0], r2(lp['D'][1]), lp['E'][0], r2(lp['E'][1]))


def _edge_fuse_kernel(e_ref, gdb_ref, ge_ref, wc, bc,
                      eo_ref, sig_ref, np_ref):
    ce = _dot(e_ref[...], wc[...]) + bc[...]
    en = ce + gdb_ref[:, :_H] + ge_ref[...]
    sig = jax.nn.sigmoid(en)
    eo_ref[...] = e_ref[...] + jnp.maximum(en, 0.0)
    sig_ref[...] = sig
    np_ref[...] = sig * gdb_ref[:, _H:]


def _edge_fuse(e, gdb, ge, wc, bc, blk=1280):
    espec = pl.BlockSpec((blk, _H), lambda i: (i, 0))
    return pl.pallas_call(
        _edge_fuse_kernel,
        grid=(_E // blk,),
        in_specs=[
            espec,
            pl.BlockSpec((blk, 2 * _H), lambda i: (i, 0)),
            espec,
            pl.BlockSpec((_H, _H), lambda i: (0, 0)),
            pl.BlockSpec((1, _H), lambda i: (0, 0)),
        ],
        out_specs=[espec, espec, espec],
        out_shape=[jax.ShapeDtypeStruct((_E, _H), _f32)] * 3,
    )(e, gdb, ge, wc, bc.reshape(1, _H))


def _h_update_kernel(h_ref, ah_ref, num_ref, den_ref, o_ref):
    o_ref[...] = h_ref[...] + jnp.maximum(
        ah_ref[...] + num_ref[...] / (den_ref[...] + 1e-6), 0.0)


def _h_update(h, ah, num, den, blk=2000):
    spec = pl.BlockSpec((blk, _H), lambda i: (i, 0))
    return pl.pallas_call(
        _h_update_kernel,
        grid=(_N // blk,),
        in_specs=[spec] * 4,
        out_specs=spec,
        out_shape=jax.ShapeDtypeStruct((_N, _H), _f32),
    )(h, ah, num, den)


def _node_ro_kernel(h_ref, w1, b1, w2, b2, w3, b3, wea, web, beb,
                    hn_ref, p1_ref, p2_ref):
    h = h_ref[...]
    t = jnp.maximum(_dot(h, w1[...]) + b1[...], 0.0)
    t = jnp.maximum(_dot(t, w2[...]) + b2[...], 0.0)
    hn_ref[...] = _dot(t, w3[...]) + b3[...]
    p1_ref[...] = _dot(h, wea[...])
    p2_ref[...] = _dot(h, web[...]) + beb[...]


def _node_ro(h, mlp_n, wea, web, beb, blk=2000):
    specs = []
    args = [h]
    for (w, b) in mlp_n:
        k, n = w.shape
        specs += [pl.BlockSpec((k, n), lambda i: (0, 0)),
                  pl.BlockSpec((1, n), lambda i: (0, 0))]
        args += [w, b.reshape(1, n)]
    specs += [pl.BlockSpec((_H, _H), lambda i: (0, 0))] * 2
    specs += [pl.BlockSpec((1, _H), lambda i: (0, 0))]
    args += [wea, web, beb.reshape(1, _H)]
    hspec = pl.BlockSpec((blk, _H), lambda i: (i, 0))
    return pl.pallas_call(
        _node_ro_kernel,
        grid=(_N // blk,),
        in_specs=[hspec] + specs,
        out_specs=[hspec, hspec, hspec],
        out_shape=[jax.ShapeDtypeStruct((_N, _H), _f32)] * 3,
    )(*args)


def _edge_mlp_kernel(g1_ref, g2_ref, w2, b2, w3, b3, o_ref):
    g = jnp.maximum(g1_ref[...] + g2_ref[...], 0.0)
    t = jnp.maximum(_dot(g, w2[...]) + b2[...], 0.0)
    o_ref[...] = _dot(t, w3[...]) + b3[...]


def _edge_mlp(g1, g2, l2, l3, blk=1280):
    w2, b2 = l2
    w3, b3 = l3
    espec = pl.BlockSpec((blk, _H), lambda i: (i, 0))
    return pl.pallas_call(
        _edge_mlp_kernel,
        grid=(_E // blk,),
        in_specs=[
            espec, espec,
            pl.BlockSpec(w2.shape, lambda i: (0, 0)),
            pl.BlockSpec((1, w2.shape[1]), lambda i: (0, 0)),
            pl.BlockSpec(w3.shape, lambda i: (0, 0)),
            pl.BlockSpec((1, w3.shape[1]), lambda i: (0, 0)),
        ],
        out_specs=espec,
        out_shape=jax.ShapeDtypeStruct((_E, _H), _f32),
    )(g1, g2, w2, b2.reshape(1, -1), w3, b3.reshape(1, -1))


_MESH = plsc.VectorSubcoreMesh(core_axis_name="c", subcore_axis_name="s")


def _make_gather2(d1, d2):
    per_w = _E // _NW
    n_chunks = per_w // _CH

    def body(t1, t2, i1_hbm, i2_hbm, o1, o2, i1_v, i2_v, r1_v, r2_v, sem):
        wid = lax.axis_index("s") * _NC + lax.axis_index("c")
        base = wid * per_w

        @pl.loop(0, n_chunks)
        def _(i):
            off = pl.multiple_of(base + i * _CH, 8)
            pltpu.sync_copy(i1_hbm.at[pl.ds(off, _CH)], i1_v)
            pltpu.sync_copy(i2_hbm.at[pl.ds(off, _CH)], i2_v)
            c1 = pltpu.async_copy(t1.at[i1_v], r1_v, sem)
            c2 = pltpu.async_copy(t2.at[i2_v], r2_v, sem)
            c1.wait()
            c2.wait()
            pltpu.sync_copy(r1_v, o1.at[pl.ds(off, _CH)])
            pltpu.sync_copy(r2_v, o2.at[pl.ds(off, _CH)])

    return pl.kernel(
        body,
        out_type=(
            jax.ShapeDtypeStruct((_E, d1), _f32),
            jax.ShapeDtypeStruct((_E, d2), _f32),
        ),
        mesh=_MESH,
        scratch_types=[
            pltpu.VMEM((_CH,), jnp.int32),
            pltpu.VMEM((_CH,), jnp.int32),
            pltpu.VMEM((_CH, d1), _f32),
            pltpu.VMEM((_CH, d2), _f32),
            pltpu.SemaphoreType.DMA,
        ],
    )


_gather_db_e = _make_gather2(2 * _H, _H)
_gather_p1_p2 = _make_gather2(_H, _H)


_NROWS = 632
_N_PAD = _NROWS * _NS


def _scatter2_body(np_hbm, sig_hbm, dst_hbm, zero_hbm, num_hbm, den_hbm,
                   idx_v, pay_v, acc_sh):
    cid = lax.axis_index("c")
    sid = lax.axis_index("s")
    nrows = _NROWS
    rows0 = sid * nrows
    per_tile = _E // _NS
    ebase = sid * per_tile

    pltpu.sync_copy(zero_hbm.at[pl.ds(rows0, nrows)],
                    acc_sh.at[pl.ds(rows0, nrows)])
    plsc.subcore_barrier()

    def scatter_from(src_hbm):
        @pl.loop(0, per_tile // _CH)
        def _(i):
            off = pl.multiple_of(ebase + i * _CH, 8)
            pltpu.sync_copy(dst_hbm.at[pl.ds(off, _CH)], idx_v)
            pltpu.sync_copy(src_hbm.at[pl.ds(off, _CH)], pay_v)
            pltpu.sync_copy(pay_v, acc_sh.at[idx_v], add=True)

    @pl.when(cid == 0)
    def _():
        scatter_from(np_hbm)

    @pl.when(cid == 1)
    def _():
        scatter_from(sig_hbm)

    plsc.subcore_barrier()

    @pl.when(cid == 0)
    def _():
        pltpu.sync_copy(acc_sh.at[pl.ds(rows0, nrows)],
                        num_hbm.at[pl.ds(rows0, nrows)])

    @pl.when(cid == 1)
    def _():
        pltpu.sync_copy(acc_sh.at[pl.ds(rows0, nrows)],
                        den_hbm.at[pl.ds(rows0, nrows)])


_scatter2 = pl.kernel(
    _scatter2_body,
    out_type=(
        jax.ShapeDtypeStruct((_N_PAD, _H), _f32),
        jax.ShapeDtypeStruct((_N_PAD, _H), _f32),
    ),
    mesh=_MESH,
    scratch_types=[
        pltpu.VMEM((_CH,), jnp.int32),
        pltpu.VMEM((_CH, _H), _f32),
        pltpu.VMEM_SHARED((_N_PAD, _H), _f32),
    ],
)


def kernel(h, e, edge_index, params):
    src = edge_index[0]
    dst = edge_index[1]
    r2 = lambda b: b.reshape(1, -1)

    h = _mm(h, params['emb_h'][0], r2(params['emb_h'][1]), blk=2000)
    e = _mm(e, params['emb_e'][0], r2(params['emb_e'][1]), blk=3200)
    zeros = jnp.zeros((_N_PAD, _H), _f32)

    for lp in params['layers']:
        ah, db, eh = _node_tf(h, lp)
        gdb, ge = _gather_db_e(db, eh, src, dst)
        e_out, sig, npay = _edge_fuse(e, gdb, ge, lp['C'][0], lp['C'][1])
        num, den = _scatter2(npay, sig, dst, zeros)
        h = _h_update(h, ah, num[:_N], den[:_N])
        e = e_out

    w1, b1 = params['mlp_e'][0]
    hn, p1, p2 = _node_ro(h, params['mlp_n'], w1[:_H], w1[_H:], b1)
    g1, g2 = _gather_p1_p2(p1, p2, src, dst)
    ef = _edge_mlp(g1, g2, params['mlp_e'][1], params['mlp_e'][2])
    return hn, ef


# speedup vs baseline: 3.3484x; 1.3349x over previous
import functools

import jax
import jax.numpy as jnp
from jax import lax
from jax.experimental import pallas as pl
from jax.experimental.pallas import tpu as pltpu
from jax.experimental.pallas import tpu_sc as plsc

_N = 10000
_E = 320000
_H = 128
_NC = 2
_NS = 16
_NW = _NC * _NS
_CH = 80

_f32 = jnp.float32


def _dot(a, b):
    return jnp.dot(a, b, preferred_element_type=_f32)


def _mm_bias_kernel(x_ref, w_ref, b_ref, o_ref):
    o_ref[...] = _dot(x_ref[...], w_ref[...]) + b_ref[...]


def _mm(x, w, b, blk):
    m, k = x.shape
    n = w.shape[1]
    return pl.pallas_call(
        _mm_bias_kernel,
        grid=(m // blk,),
        in_specs=[
            pl.BlockSpec((blk, k), lambda i: (i, 0)),
            pl.BlockSpec((k, n), lambda i: (0, 0)),
            pl.BlockSpec((1, n), lambda i: (0, 0)),
        ],
        out_specs=pl.BlockSpec((blk, n), lambda i: (i, 0)),
        out_shape=jax.ShapeDtypeStruct((m, n), _f32),
    )(x, w, b)


def _node_tf_kernel(h_ref, wa, ba, wb, bb, wd, bd, we, be,
                    ah_ref, db_ref, eh_ref):
    h = h_ref[...]
    ah_ref[...] = _dot(h, wa[...]) + ba[...]
    db_ref[:, :_H] = _dot(h, wd[...]) + bd[...]
    db_ref[:, _H:] = _dot(h, wb[...]) + bb[...]
    eh_ref[...] = _dot(h, we[...]) + be[...]


def _node_tf(h, lp, blk=2000):
    wspec = pl.BlockSpec((_H, _H), lambda i: (0, 0))
    bspec = pl.BlockSpec((1, _H), lambda i: (0, 0))
    r2 = lambda b: b.reshape(1, _H)
    return pl.pallas_call(
        _node_tf_kernel,
        grid=(_N // blk,),
        in_specs=[pl.BlockSpec((blk, _H), lambda i: (i, 0))]
        + [wspec, bspec] * 4,
        out_specs=[
            pl.BlockSpec((blk, _H), lambda i: (i, 0)),
            pl.BlockSpec((blk, 2 * _H), lambda i: (i, 0)),
            pl.BlockSpec((blk, _H), lambda i: (i, 0)),
        ],
        out_shape=[
            jax.ShapeDtypeStruct((_N, _H), _f32),
            jax.ShapeDtypeStruct((_N, 2 * _H), _f32),
            jax.ShapeDtypeStruct((_N, _H), _f32),
        ],
    )(h, lp['A'][0], r2(lp['A'][1]), lp['B'][0], r2(lp['B'][1]),
      lp['D'][0], r2(lp['D'][1]), lp['E'][0], r2(lp['E'][1]))


def _edge_fuse_kernel(e_ref, gdb_ref, ge_ref, wc, bc,
                      eo_ref, sig_ref, np_ref):
    ce = _dot(e_ref[...], wc[...]) + bc[...]
    en = ce + gdb_ref[:, :_H] + ge_ref[...]
    sig = jax.nn.sigmoid(en)
    eo_ref[...] = e_ref[...] + jnp.maximum(en, 0.0)
    sig_ref[...] = sig
    np_ref[...] = sig * gdb_ref[:, _H:]


def _edge_fuse(e, gdb, ge, wc, bc, blk=1280):
    espec = pl.BlockSpec((blk, _H), lambda i: (i, 0))
    return pl.pallas_call(
        _edge_fuse_kernel,
        grid=(_E // blk,),
        in_specs=[
            espec,
            pl.BlockSpec((blk, 2 * _H), lambda i: (i, 0)),
            espec,
            pl.BlockSpec((_H, _H), lambda i: (0, 0)),
            pl.BlockSpec((1, _H), lambda i: (0, 0)),
        ],
        out_specs=[espec, espec, espec],
        out_shape=[jax.ShapeDtypeStruct((_E, _H), _f32)] * 3,
    )(e, gdb, ge, wc, bc.reshape(1, _H))


def _h_update_kernel(h_ref, ah_ref, num_ref, den_ref, o_ref):
    o_ref[...] = h_ref[...] + jnp.maximum(
        ah_ref[...] + num_ref[...] / (den_ref[...] + 1e-6), 0.0)


def _h_update(h, ah, num, den, blk=2000):
    spec = pl.BlockSpec((blk, _H), lambda i: (i, 0))
    return pl.pallas_call(
        _h_update_kernel,
        grid=(_N // blk,),
        in_specs=[spec] * 4,
        out_specs=spec,
        out_shape=jax.ShapeDtypeStruct((_N, _H), _f32),
    )(h, ah, num, den)


def _node_ro_kernel(h_ref, w1, b1, w2, b2, w3, b3, wea, web, beb,
                    hn_ref, p1_ref, p2_ref):
    h = h_ref[...]
    t = jnp.maximum(_dot(h, w1[...]) + b1[...], 0.0)
    t = jnp.maximum(_dot(t, w2[...]) + b2[...], 0.0)
    hn_ref[...] = _dot(t, w3[...]) + b3[...]
    p1_ref[...] = _dot(h, wea[...])
    p2_ref[...] = _dot(h, web[...]) + beb[...]


def _node_ro(h, mlp_n, wea, web, beb, blk=2000):
    specs = []
    args = [h]
    for (w, b) in mlp_n:
        k, n = w.shape
        specs += [pl.BlockSpec((k, n), lambda i: (0, 0)),
                  pl.BlockSpec((1, n), lambda i: (0, 0))]
        args += [w, b.reshape(1, n)]
    specs += [pl.BlockSpec((_H, _H), lambda i: (0, 0))] * 2
    specs += [pl.BlockSpec((1, _H), lambda i: (0, 0))]
    args += [wea, web, beb.reshape(1, _H)]
    hspec = pl.BlockSpec((blk, _H), lambda i: (i, 0))
    return pl.pallas_call(
        _node_ro_kernel,
        grid=(_N // blk,),
        in_specs=[hspec] + specs,
        out_specs=[hspec, hspec, hspec],
        out_shape=[jax.ShapeDtypeStruct((_N, _H), _f32)] * 3,
    )(*args)


def _edge_mlp_kernel(g1_ref, g2_ref, w2, b2, w3, b3, o_ref):
    g = jnp.maximum(g1_ref[...] + g2_ref[...], 0.0)
    t = jnp.maximum(_dot(g, w2[...]) + b2[...], 0.0)
    o_ref[...] = _dot(t, w3[...]) + b3[...]


def _edge_mlp(g1, g2, l2, l3, blk=1280):
    w2, b2 = l2
    w3, b3 = l3
    espec = pl.BlockSpec((blk, _H), lambda i: (i, 0))
    return pl.pallas_call(
        _edge_mlp_kernel,
        grid=(_E // blk,),
        in_specs=[
            espec, espec,
            pl.BlockSpec(w2.shape, lambda i: (0, 0)),
            pl.BlockSpec((1, w2.shape[1]), lambda i: (0, 0)),
            pl.BlockSpec(w3.shape, lambda i: (0, 0)),
            pl.BlockSpec((1, w3.shape[1]), lambda i: (0, 0)),
        ],
        out_specs=espec,
        out_shape=jax.ShapeDtypeStruct((_E, _H), _f32),
    )(g1, g2, w2, b2.reshape(1, -1), w3, b3.reshape(1, -1))


_MESH = plsc.VectorSubcoreMesh(core_axis_name="c", subcore_axis_name="s")


def _make_gather2(d1, d2):
    per_w = _E // _NW
    n_chunks = per_w // _CH

    def body(t1, t2, i1_hbm, i2_hbm, o1, o2,
             i1_v, i2_v, r1_v, r2_v, si0, si1, sg0, sg1, so0, so1):
        wid = lax.axis_index("s") * _NC + lax.axis_index("c")
        base = wid * per_w
        si = (si0, si1)
        sg = (sg0, sg1)
        so = (so0, so1)

        def idx_load(i, b, sem_fn=pltpu.async_copy):
            off = pl.multiple_of(base + i * _CH, 8)
            sem_fn(i1_hbm.at[pl.ds(off, _CH)], i1_v.at[b], si[b])
            sem_fn(i2_hbm.at[pl.ds(off, _CH)], i2_v.at[b], si[b])

        def idx_wait(i, b):
            off = pl.multiple_of(base + i * _CH, 8)
            pltpu.make_async_copy(
                i1_hbm.at[pl.ds(off, _CH)], i1_v.at[b], si[b]).wait()
            pltpu.make_async_copy(
                i2_hbm.at[pl.ds(off, _CH)], i2_v.at[b], si[b]).wait()

        def gather_issue(b):
            pltpu.async_copy(t1.at[i1_v.at[b]], r1_v.at[b], sg[b])
            pltpu.async_copy(t2.at[i2_v.at[b]], r2_v.at[b], sg[b])

        def gather_wait(b):
            pltpu.make_async_copy(t1.at[i1_v.at[b]], r1_v.at[b], sg[b]).wait()
            pltpu.make_async_copy(t2.at[i2_v.at[b]], r2_v.at[b], sg[b]).wait()

        def write_issue(i, b):
            off = pl.multiple_of(base + i * _CH, 8)
            pltpu.async_copy(r1_v.at[b], o1.at[pl.ds(off, _CH)], so[b])
            pltpu.async_copy(r2_v.at[b], o2.at[pl.ds(off, _CH)], so[b])

        def write_wait(i, b):
            off = pl.multiple_of(base + i * _CH, 8)
            pltpu.make_async_copy(
                r1_v.at[b], o1.at[pl.ds(off, _CH)], so[b]).wait()
            pltpu.make_async_copy(
                r2_v.at[b], o2.at[pl.ds(off, _CH)], so[b]).wait()

        def maybe(cond, fn):
            if cond is True:
                fn()
            elif cond is not False:
                pl.when(cond)(fn)

        def stage(i, b, has_prev, has_prev2, has_next):
            maybe(has_prev2, lambda: write_wait(i - 2, b))
            idx_wait(i, b)
            gather_issue(b)

            def drain_prev():
                gather_wait(1 - b)
                write_issue(i - 1, 1 - b)
            maybe(has_prev, drain_prev)
            maybe(has_next, lambda: idx_load(i + 1, 1 - b))

        idx_load(0, 0)

        @pl.loop(0, n_chunks // 2)
        def _(j):
            i0 = j * 2
            stage(i0, 0, j > 0, j > 0, True)
            stage(i0 + 1, 1, True, j > 0, i0 + 2 < n_chunks)

        last = n_chunks - 1
        if n_chunks % 2 == 1:
            write_wait(last - 2, 0)
            idx_wait(last, 0)
            gather_issue(0)
            gather_wait(1)
            write_issue(last - 1, 1)
            gather_wait(0)
            write_issue(last, 0)
            write_wait(last - 1, 1)
            write_wait(last, 0)
        else:
            gather_wait(1)
            write_issue(last, 1)
            write_wait(last - 1, 0)
            write_wait(last, 1)

    return pl.kernel(
        body,
        out_type=(
            jax.ShapeDtypeStruct((_E, d1), _f32),
            jax.ShapeDtypeStruct((_E, d2), _f32),
        ),
        mesh=_MESH,
        scratch_types=[
            pltpu.VMEM((2, _CH), jnp.int32),
            pltpu.VMEM((2, _CH), jnp.int32),
            pltpu.VMEM((2, _CH, d1), _f32),
            pltpu.VMEM((2, _CH, d2), _f32),
        ] + [pltpu.SemaphoreType.DMA] * 6,
    )


_gather_db_e = _make_gather2(2 * _H, _H)
_gather_p1_p2 = _make_gather2(_H, _H)


_NROWS = 632
_N_PAD = _NROWS * _NS


def _scatter2_body(np_hbm, sig_hbm, dst_hbm, zero_hbm, num_hbm, den_hbm,
                   idx_v, pay_v, acc_sh, si0, si1, sp0, sp1, ss0, ss1):
    cid = lax.axis_index("c")
    sid = lax.axis_index("s")
    nrows = _NROWS
    rows0 = sid * nrows
    per_tile = _E // _NS
    ebase = sid * per_tile

    pltpu.sync_copy(zero_hbm.at[pl.ds(rows0, nrows)],
                    acc_sh.at[pl.ds(rows0, nrows)])
    plsc.subcore_barrier()

    n_chunks = per_tile // _CH

    def scatter_from(src_hbm):
        si = (si0, si1)
        sp = (sp0, sp1)
        ss = (ss0, ss1)

        def load(i, b):
            off = pl.multiple_of(ebase + i * _CH, 8)
            pltpu.async_copy(dst_hbm.at[pl.ds(off, _CH)], idx_v.at[b], si[b])
            pltpu.async_copy(src_hbm.at[pl.ds(off, _CH)], pay_v.at[b], sp[b])

        def load_wait(i, b):
            off = pl.multiple_of(ebase + i * _CH, 8)
            pltpu.make_async_copy(
                dst_hbm.at[pl.ds(off, _CH)], idx_v.at[b], si[b]).wait()
            pltpu.make_async_copy(
                src_hbm.at[pl.ds(off, _CH)], pay_v.at[b], sp[b]).wait()

        def scat_issue(b):
            pltpu.async_copy(pay_v.at[b], acc_sh.at[idx_v.at[b]], ss[b],
                             add=True)

        def scat_wait(b):
            pltpu.make_async_copy(pay_v.at[b], acc_sh.at[idx_v.at[b]],
                                  ss[b]).wait()

        def maybe(cond, fn):
            if cond is True:
                fn()
            elif cond is not False:
                pl.when(cond)(fn)

        def stage(i, b, has_prev, has_next):
            load_wait(i, b)
            scat_issue(b)
            maybe(has_prev, lambda: scat_wait(1 - b))
            maybe(has_next, lambda: load(i + 1, 1 - b))

        load(0, 0)

        @pl.loop(0, n_chunks // 2)
        def _(j):
            i0 = j * 2
            stage(i0, 0, j > 0, True)
            stage(i0 + 1, 1, True, i0 + 2 < n_chunks)

        scat_wait(1)

    @pl.when(cid == 0)
    def _():
        scatter_from(np_hbm)

    @pl.when(cid == 1)
    def _():
        scatter_from(sig_hbm)

    plsc.subcore_barrier()

    @pl.when(cid == 0)
    def _():
        pltpu.sync_copy(acc_sh.at[pl.ds(rows0, nrows)],
                        num_hbm.at[pl.ds(rows0, nrows)])

    @pl.when(cid == 1)
    def _():
        pltpu.sync_copy(acc_sh.at[pl.ds(rows0, nrows)],
                        den_hbm.at[pl.ds(rows0, nrows)])


_scatter2 = pl.kernel(
    _scatter2_body,
    out_type=(
        jax.ShapeDtypeStruct((_N_PAD, _H), _f32),
        jax.ShapeDtypeStruct((_N_PAD, _H), _f32),
    ),
    mesh=_MESH,
    scratch_types=[
        pltpu.VMEM((2, _CH), jnp.int32),
        pltpu.VMEM((2, _CH, _H), _f32),
        pltpu.VMEM_SHARED((_N_PAD, _H), _f32),
    ] + [pltpu.SemaphoreType.DMA] * 6,
)


def kernel(h, e, edge_index, params):
    src = edge_index[0]
    dst = edge_index[1]
    r2 = lambda b: b.reshape(1, -1)

    h = _mm(h, params['emb_h'][0], r2(params['emb_h'][1]), blk=2000)
    e = _mm(e, params['emb_e'][0], r2(params['emb_e'][1]), blk=3200)
    zeros = jnp.zeros((_N_PAD, _H), _f32)

    for lp in params['layers']:
        ah, db, eh = _node_tf(h, lp)
        gdb, ge = _gather_db_e(db, eh, src, dst)
        e_out, sig, npay = _edge_fuse(e, gdb, ge, lp['C'][0], lp['C'][1])
        num, den = _scatter2(npay, sig, dst, zeros)
        h = _h_update(h, ah, num[:_N], den[:_N])
        e = e_out

    w1, b1 = params['mlp_e'][0]
    hn, p1, p2 = _node_ro(h, params['mlp_n'], w1[:_H], w1[_H:], b1)
    g1, g2 = _gather_p1_p2(p1, p2, src, dst)
    ef = _edge_mlp(g1, g2, params['mlp_e'][1], params['mlp_e'][2])
    return hn, ef
